# baseline (speedup 1.0000x reference)
import functools

import jax
import jax.numpy as jnp
from jax import lax
from jax.experimental import pallas as pl
from jax.experimental.pallas import tpu as pltpu
from jax.experimental.pallas import tpu_sc as plsc

NC = 2
NS = 16
L = 16
NW = NC * NS

N = 100000
CLS = 5
NP = 3 * N
NY = 1024
YC = 19

G16 = N // L
GPW = 196
LAST_BASE = G16 - GPW
ZCHUNK = GPW * L
XCHUNK = ZCHUNK * CLS

PG = NP // 48
PGPW = 391
PGPW_LAST = PG - 15 * PGPW
PCHUNK = PGPW * 48
PCHUNK_BASE = PGPW_LAST * 48
PEXTRA = PCHUNK - PCHUNK_BASE

YROWS = NY // NW
YIN = YROWS * YC
YOUT = YROWS * 4

_mesh = plsc.VectorSubcoreMesh(core_axis_name="c", subcore_axis_name="s")


@functools.partial(
    pl.kernel,
    out_type=[
        jax.ShapeDtypeStruct((NY * 4,), jnp.float32),
        jax.ShapeDtypeStruct((N * CLS,), jnp.float32),
        jax.ShapeDtypeStruct((L,), jnp.float32),
        jax.ShapeDtypeStruct((N,), jnp.int32),
    ],
    mesh=_mesh,
    compiler_params=pltpu.CompilerParams(needs_layout_passes=False),
    scratch_types=[
        pltpu.VMEM((ZCHUNK,), jnp.int32),
        pltpu.VMEM((L,), jnp.int32),
        pltpu.VMEM((XCHUNK,), jnp.float32),
        pltpu.VMEM((YIN,), jnp.float32),
        pltpu.VMEM((YOUT,), jnp.float32),
        pltpu.VMEM((PCHUNK,), jnp.float32),
        pltpu.VMEM((3 * L,), jnp.float32),
        pltpu.VMEM((L,), jnp.float32),
        pltpu.VMEM((NS * L,), jnp.float32),
        pltpu.VMEM((L,), jnp.float32),
        pltpu.VMEM_SHARED((NS * L,), jnp.float32),
    ],
)
def _sc_transform(y_hbm, z_hbm, pos_hbm,
                  ysel_out, x_out, mean_out, z_out,
                  zbuf, z2tmp, xbuf, ybuf, youtbuf, posbuf,
                  accbuf, partialbuf, sumbuf, meanbuf, shared):
    c = lax.axis_index("c")
    s = lax.axis_index("s")
    w = c * NS + s
    ii = lax.iota(jnp.int32, L)
    zeros_i = ii - ii
    ones_i = zeros_i + 1
    fours_i = zeros_i + 4
    zeros_f = zeros_i.astype(jnp.float32)
    ones_f = zeros_f + 1.0

    base_g = jnp.minimum(w * GPW, LAST_BASE)
    zoff = base_g * L
    pltpu.sync_copy(z_hbm.at[pl.ds(zoff, ZCHUNK)], zbuf)
    pltpu.sync_copy(zbuf, z_out.at[pl.ds(zoff, ZCHUNK)])

    def xbody(g, carry):
        zv = zbuf[pl.ds(g * L, L)]
        z2 = zv - ones_i
        z2 = jnp.where(z2 != zeros_i, z2 - fours_i, z2)
        z2tmp[...] = z2
        jj = lax.iota(jnp.int32, L)
        for t in range(CLS):
            rp = (jj + 16 * t) // 5
            cp = (jj + 16 * t) % 5
            gt = plsc.load_gather(z2tmp, [rp])
            v = jnp.where(gt == cp, ones_f, zeros_f)
            xbuf[pl.ds(g * (L * CLS) + t * L, L)] = v
        return carry

    lax.fori_loop(0, GPW, xbody, 0)
    pltpu.sync_copy(xbuf, x_out.at[pl.ds(base_g * (L * CLS), XCHUNK)])

    pltpu.sync_copy(y_hbm.at[pl.ds(w * YIN, YIN)], ybuf)
    for t in range(YOUT // L):
        p = ii + 16 * t
        m4 = p % 4
        col = 2 * m4 + jnp.where(m4 == 3, ones_i, zeros_i)
        src = (p // 4) * YC + col
        gv = plsc.load_gather(ybuf, [src])
        youtbuf[pl.ds(t * L, L)] = gv
    pltpu.sync_copy(youtbuf, ysel_out.at[pl.ds(w * YOUT, YOUT)])

    @pl.when(c == 0)
    def _pos_partials():
        poff = s * PCHUNK
        pltpu.sync_copy(pos_hbm.at[pl.ds(poff, PCHUNK_BASE)],
                        posbuf.at[pl.ds(0, PCHUNK_BASE)])

        @pl.when(s < NS - 1)
        def _tail():
            pltpu.sync_copy(
                pos_hbm.at[pl.ds(poff + PCHUNK_BASE, PEXTRA)],
                posbuf.at[pl.ds(PCHUNK_BASE, PEXTRA)])

        for k in range(3):
            accbuf[pl.ds(k * L, L)] = zeros_f

        ng = jnp.where(s == NS - 1, PGPW_LAST, PGPW)

        def pbody(i, carry):
            off = i * 48
            for k in range(3):
                plsc.addupdate(accbuf.at[pl.ds(k * L, L)],
                               posbuf[pl.ds(off + k * L, L)])
            return carry

        lax.fori_loop(0, ng, pbody, 0)
        pvec = zeros_f
        for j in range(3):
            tj = jnp.float32(0.0)
            for k in range(3):
                mk = ((ii + 16 * k) % 3) == j
                tj = tj + jnp.sum(
                    jnp.where(mk, accbuf[pl.ds(k * L, L)], zeros_f))
            tj_v = jnp.full((L,), tj, jnp.float32)
            pvec = pvec + jnp.where(ii == j, tj_v, zeros_f)
        partialbuf[...] = pvec
        pltpu.sync_copy(partialbuf, shared.at[pl.ds(s * L, L)])

    plsc.subcore_barrier()

    @pl.when(jnp.logical_and(c == 0, s == 0))
    def _finalize_mean():
        pltpu.sync_copy(shared, sumbuf)
        meanbuf[...] = zeros_f

        def rbody(i, carry):
            plsc.addupdate(meanbuf.at[pl.ds(0, L)],
                           sumbuf[pl.ds(i * L, L)])
            return carry

        lax.fori_loop(0, NS, rbody, 0)
        meanbuf[...] = meanbuf[...] * (zeros_f + jnp.float32(1.0 / N))
        pltpu.sync_copy(meanbuf, mean_out)


def kernel(y, z, pos):
    ysel_f, x_f, mean16, z_orig = _sc_transform(
        y.reshape(-1), z, pos.reshape(-1))
    y_sel = ysel_f.reshape(NY, 4)
    x = x_f.reshape(N, CLS)
    mean = mean16[:3].reshape(1, 3)
    return (y_sel, x, mean, z_orig)


# speedup vs baseline: 1.0602x; 1.0602x over previous
import functools

import jax
import jax.numpy as jnp
from jax import lax
from jax.experimental import pallas as pl
from jax.experimental.pallas import tpu as pltpu
from jax.experimental.pallas import tpu_sc as plsc

NC = 2
NS = 16
L = 16
NW = NC * NS

N = 100000
CLS = 5
NP = 3 * N
NY = 1024
YC = 19

G16 = N // L
GPW = 196
LAST_BASE = G16 - GPW
ZCHUNK = GPW * L
XCHUNK = ZCHUNK * CLS

PG = NP // 48
PGPW = 391
PGPW_LAST = PG - 15 * PGPW
PCHUNK = PGPW * 48
PCHUNK_BASE = PGPW_LAST * 48
PEXTRA = PCHUNK - PCHUNK_BASE
PTAIL_G = PEXTRA // 48

YROWS = NY // NW
YIN = YROWS * YC
YOUT = YROWS * 4

_mesh = plsc.VectorSubcoreMesh(core_axis_name="c", subcore_axis_name="s")


@functools.partial(
    pl.kernel,
    out_type=[
        jax.ShapeDtypeStruct((NY * 4,), jnp.float32),
        jax.ShapeDtypeStruct((N * CLS,), jnp.float32),
        jax.ShapeDtypeStruct((L,), jnp.float32),
        jax.ShapeDtypeStruct((N,), jnp.int32),
    ],
    mesh=_mesh,
    compiler_params=pltpu.CompilerParams(needs_layout_passes=False),
    scratch_types=[
        pltpu.VMEM((ZCHUNK,), jnp.int32),
        pltpu.VMEM((XCHUNK,), jnp.float32),
        pltpu.VMEM((YIN,), jnp.float32),
        pltpu.VMEM((YOUT,), jnp.float32),
        pltpu.VMEM((PCHUNK,), jnp.float32),
        pltpu.VMEM((3 * L,), jnp.float32),
        pltpu.VMEM((L,), jnp.float32),
        pltpu.VMEM((NS * L,), jnp.float32),
        pltpu.VMEM((L,), jnp.float32),
        pltpu.VMEM_SHARED((NS * L,), jnp.float32),
        pltpu.SemaphoreType.DMA,
        pltpu.SemaphoreType.DMA,
        pltpu.SemaphoreType.DMA,
        pltpu.SemaphoreType.DMA,
        pltpu.SemaphoreType.DMA,
        pltpu.SemaphoreType.DMA,
    ],
)
def _sc_transform(y_hbm, z_hbm, pos_hbm,
                  ysel_out, x_out, mean_out, z_out,
                  zbuf, xbuf, ybuf, youtbuf, posbuf,
                  accbuf, partialbuf, sumbuf, meanbuf, shared,
                  sem_zin, sem_zout, sem_yin, sem_pos, sem_pos2, sem_xout):
    c = lax.axis_index("c")
    s = lax.axis_index("s")
    w = c * NS + s
    ii = lax.iota(jnp.int32, L)
    zeros_i = ii - ii
    ones_i = zeros_i + 1
    fours_i = zeros_i + 4
    fives_u = plsc.bitcast(zeros_i + CLS, jnp.uint32)
    zeros_f = zeros_i.astype(jnp.float32)
    ones_f = zeros_f + 1.0
    lane5 = ii * CLS

    base_g = jnp.minimum(w * GPW, LAST_BASE)
    zoff = base_g * L
    d_zin = pltpu.async_copy(z_hbm.at[pl.ds(zoff, ZCHUNK)], zbuf, sem_zin)
    d_yin = pltpu.async_copy(y_hbm.at[pl.ds(w * YIN, YIN)], ybuf, sem_yin)

    @pl.when(c == 0)
    def _pos_loads():
        poff = s * PCHUNK
        pltpu.async_copy(pos_hbm.at[pl.ds(poff, PCHUNK_BASE)],
                         posbuf.at[pl.ds(0, PCHUNK_BASE)], sem_pos)

        @pl.when(s < NS - 1)
        def _tail_load():
            pltpu.async_copy(pos_hbm.at[pl.ds(poff + PCHUNK_BASE, PEXTRA)],
                             posbuf.at[pl.ds(PCHUNK_BASE, PEXTRA)], sem_pos2)

    def mbody(m, carry):
        xbuf[pl.ds(m * L, L)] = zeros_f
        return carry

    lax.fori_loop(0, XCHUNK // L, mbody, 0, unroll=16)

    d_zin.wait()
    d_zout = pltpu.async_copy(zbuf, z_out.at[pl.ds(zoff, ZCHUNK)], sem_zout)
    for g in range(GPW):
        zv = zbuf[pl.ds(g * L, L)]
        z2 = zv - ones_i
        z2 = jnp.where(z2 != zeros_i, z2 - fours_i, z2)
        idx = z2 + (lane5 + g * (L * CLS))
        mask = plsc.bitcast(z2, jnp.uint32) < fives_u
        plsc.store_scatter(xbuf, [idx], ones_f, mask=mask)
    d_xout = pltpu.async_copy(
        xbuf, x_out.at[pl.ds(base_g * (L * CLS), XCHUNK)], sem_xout)

    d_yin.wait()
    for t in range(YOUT // L):
        p = ii + 16 * t
        m4 = p % 4
        col = 2 * m4 + jnp.where(m4 == 3, ones_i, zeros_i)
        src = (p // 4) * YC + col
        youtbuf[pl.ds(t * L, L)] = plsc.load_gather(ybuf, [src])
    pltpu.sync_copy(youtbuf, ysel_out.at[pl.ds(w * YOUT, YOUT)])

    @pl.when(c == 0)
    def _pos_partials():
        pltpu.make_async_copy(pos_hbm.at[pl.ds(0, PCHUNK_BASE)],
                              posbuf.at[pl.ds(0, PCHUNK_BASE)],
                              sem_pos).wait()
        for k in range(3):
            accbuf[pl.ds(k * L, L)] = zeros_f

        @pl.when(s < NS - 1)
        def _tail_sum():
            pltpu.make_async_copy(
                pos_hbm.at[pl.ds(0, PEXTRA)],
                posbuf.at[pl.ds(PCHUNK_BASE, PEXTRA)], sem_pos2).wait()
            for gg in range(PTAIL_G):
                off = PCHUNK_BASE + gg * 48
                for k in range(3):
                    plsc.addupdate(accbuf.at[pl.ds(k * L, L)],
                                   posbuf[pl.ds(off + k * L, L)])

        def pbody(i, acc):
            a0, a1, a2 = acc
            off = i * 48
            a0 = a0 + posbuf[pl.ds(off, L)]
            a1 = a1 + posbuf[pl.ds(off + L, L)]
            a2 = a2 + posbuf[pl.ds(off + 2 * L, L)]
            return (a0, a1, a2)

        accs = lax.fori_loop(0, PGPW_LAST, pbody, (zeros_f, zeros_f, zeros_f),
                             unroll=8)
        accs = [accs[k] + accbuf[pl.ds(k * L, L)] for k in range(3)]
        pvec = zeros_f
        for j in range(3):
            tj = jnp.float32(0.0)
            for k in range(3):
                mk = ((ii + 16 * k) % 3) == j
                tj = tj + jnp.sum(jnp.where(mk, accs[k], zeros_f))
            tj_v = jnp.full((L,), tj, jnp.float32)
            pvec = pvec + jnp.where(ii == j, tj_v, zeros_f)
        partialbuf[...] = pvec
        pltpu.sync_copy(partialbuf, shared.at[pl.ds(s * L, L)])

    plsc.subcore_barrier()

    @pl.when(jnp.logical_and(c == 0, s == 0))
    def _finalize_mean():
        pltpu.sync_copy(shared, sumbuf)
        tot = zeros_f
        for i in range(NS):
            tot = tot + sumbuf[pl.ds(i * L, L)]
        meanbuf[...] = tot * (zeros_f + jnp.float32(1.0 / N))
        pltpu.sync_copy(meanbuf, mean_out)

    d_zout.wait()
    d_xout.wait()


def kernel(y, z, pos):
    ysel_f, x_f, mean16, z_orig = _sc_transform(
        y.reshape(-1), z, pos.reshape(-1))
    y_sel = ysel_f.reshape(NY, 4)
    x = x_f.reshape(N, CLS)
    mean = mean16[:3].reshape(1, 3)
    return (y_sel, x, mean, z_orig)
